# baseline (speedup 1.0000x reference)
import functools
import math

import jax
import jax.numpy as jnp
from jax import lax
from jax.experimental import pallas as pl
from jax.experimental.pallas import tpu as pltpu
from jax.experimental.pallas import tpu_sc as plsc


def _make_sc_kernel(B, S, V, D):
    info = plsc.get_sparse_core_info()
    NC, NS, L = info.num_cores, info.num_subcores, info.num_lanes
    NW = NC * NS
    NH = 2
    HD = D // NH
    n_groups = NW // NH
    assert S % n_groups == 0
    pos_per_w = S // n_groups
    P = 8
    n_chunks = pos_per_w // P
    NBUF = 4
    scale = jnp.float32(math.sqrt(D))

    mesh = plsc.VectorSubcoreMesh(core_axis_name="c", subcore_axis_name="s")

    @functools.partial(
        pl.kernel,
        out_type=jax.ShapeDtypeStruct((B, S, D), jnp.float32),
        mesh=mesh,
        scratch_types=[
            pltpu.VMEM((B, pos_per_w), jnp.int32),
            pltpu.VMEM((NBUF, B, P, HD), jnp.float32),
            pltpu.VMEM((NBUF, P, HD), jnp.float32),
        ] + [pltpu.SemaphoreType.DMA] * 13,
    )
    def emb_kernel(x_hbm, table_hbm, pos_hbm, out_hbm, idx_v, rows_v, pos_v,
                   gsem0, gsem1, gsem2, gsem3, osem0, osem1, osem2, osem3,
                   psem0, psem1, psem2, psem3, isem):
        wid = lax.axis_index("s") * NC + lax.axis_index("c")
        grp = wid // NH
        col0 = (wid % NH) * HD
        base = grp * pos_per_w
        gsems = (gsem0, gsem1, gsem2, gsem3)
        osems = (osem0, osem1, osem2, osem3)
        psems = (psem0, psem1, psem2, psem3)

        idx_copies = [
            pltpu.async_copy(x_hbm.at[b, pl.ds(base, pos_per_w)], idx_v.at[b],
                             isem)
            for b in range(B)
        ]

        def start_pos(ch):
            par = ch % NBUF
            return pltpu.async_copy(
                pos_hbm.at[pl.ds(base + ch * P, P), pl.ds(col0, HD)],
                pos_v.at[par], psems[par])

        def start_gathers(ch):
            par = ch % NBUF
            return [
                pltpu.async_copy(
                    table_hbm.at[idx_v.at[b, pl.ds(ch * P, P)],
                                 pl.ds(col0, HD)],
                    rows_v.at[par, b], gsems[par])
                for b in range(B)
            ]

        pos_copies = {ch: start_pos(ch) for ch in range(3)}
        for c in idx_copies:
            c.wait()
        gather_copies = {ch: start_gathers(ch) for ch in range(3)}
        wb_copies = {}

        for ch in range(n_chunks):
            par = ch % NBUF
            pos_copies[ch].wait()
            for c in gather_copies[ch]:
                c.wait()

            @plsc.parallel_loop(0, P * 4)
            def body(i, par=par):
                r = i // 4
                q = i % 4
                for k in range(8):
                    cb = q * 8 + k
                    sl = pl.ds(cb * L, L)
                    pv = pos_v[par, r, sl]
                    for b in range(B):
                        rows_v[par, b, r, sl] = rows_v[par, b, r, sl] * scale + pv

            wb_copies[ch] = [
                pltpu.async_copy(
                    rows_v.at[par, b],
                    out_hbm.at[b, pl.ds(base + ch * P, P), pl.ds(col0, HD)],
                    osems[par])
                for b in range(B)
            ]
            if ch + 3 < n_chunks:
                if ch >= 1:
                    for c in wb_copies[ch - 1]:
                        c.wait()
                gather_copies[ch + 3] = start_gathers(ch + 3)
                pos_copies[ch + 3] = start_pos(ch + 3)

        for ch in range(n_chunks - 4, n_chunks):
            for c in wb_copies[ch]:
                c.wait()

    return emb_kernel


def kernel(x, emb_table, pos_enc):
    B, S = x.shape
    V, D = emb_table.shape
    x = x.astype(jnp.int32)
    emb = _make_sc_kernel(B, S, V, D)
    return emb(x, emb_table, pos_enc)


# speedup vs baseline: 1.0908x; 1.0908x over previous
import functools
import math

import jax
import jax.numpy as jnp
from jax import lax
from jax.experimental import pallas as pl
from jax.experimental.pallas import tpu as pltpu
from jax.experimental.pallas import tpu_sc as plsc


def _make_sc_kernel(B, S, V, D):
    info = plsc.get_sparse_core_info()
    NC, NS, L = info.num_cores, info.num_subcores, info.num_lanes
    NW = NC * NS
    assert S % NW == 0
    pos_per_w = S // NW
    P = 8
    n_chunks = pos_per_w // P
    NBUF = 3
    scale = jnp.float32(math.sqrt(D))
    vecs_per_row = D // L

    mesh = plsc.VectorSubcoreMesh(core_axis_name="c", subcore_axis_name="s")

    @functools.partial(
        pl.kernel,
        out_type=jax.ShapeDtypeStruct((B, S, D), jnp.float32),
        mesh=mesh,
        scratch_types=[
            pltpu.VMEM((B, pos_per_w), jnp.int32),
            pltpu.VMEM((NBUF, B, P, D), jnp.float32),
            pltpu.VMEM((NBUF, P, D), jnp.float32),
        ] + [pltpu.SemaphoreType.DMA] * 10,
    )
    def emb_kernel(x_hbm, table_hbm, pos_hbm, out_hbm, idx_v, rows_v, pos_v,
                   gsem0, gsem1, gsem2, osem0, osem1, osem2, psem0, psem1, psem2,
                   isem):
        wid = lax.axis_index("s") * NC + lax.axis_index("c")
        base = wid * pos_per_w
        gsems = (gsem0, gsem1, gsem2)
        osems = (osem0, osem1, osem2)
        psems = (psem0, psem1, psem2)

        idx_copies = [
            pltpu.async_copy(x_hbm.at[b, pl.ds(base, pos_per_w)], idx_v.at[b],
                             isem)
            for b in range(B)
        ]

        def start_pos(ch):
            par = ch % NBUF
            return pltpu.async_copy(
                pos_hbm.at[pl.ds(base + ch * P, P)], pos_v.at[par],
                psems[par])

        def start_gathers(ch):
            par = ch % NBUF
            return [
                pltpu.async_copy(
                    table_hbm.at[idx_v.at[b, pl.ds(ch * P, P)]],
                    rows_v.at[par, b], gsems[par])
                for b in range(B)
            ]

        pos_copies = {0: start_pos(0), 1: start_pos(1)}
        for c in idx_copies:
            c.wait()
        gather_copies = {0: start_gathers(0), 1: start_gathers(1)}
        wb_copies = {}

        for ch in range(n_chunks):
            par = ch % NBUF
            pos_copies[ch].wait()
            for c in gather_copies[ch]:
                c.wait()

            if ch + 2 < n_chunks:
                pos_copies[ch + 2] = start_pos(ch + 2)

            @plsc.parallel_loop(0, P * 16)
            def body(i, par=par):
                r = i // 16
                g = i % 16
                for k in range(4):
                    cb = g * 4 + k
                    sl = pl.ds(cb * L, L)
                    pv = pos_v[par, r, sl]
                    for b in range(B):
                        rows_v[par, b, r, sl] = rows_v[par, b, r, sl] * scale + pv

            wb_copies[ch] = [
                pltpu.async_copy(
                    rows_v.at[par, b], out_hbm.at[b, pl.ds(base + ch * P, P)],
                    osems[par])
                for b in range(B)
            ]
            if ch >= 1:
                for c in wb_copies[ch - 1]:
                    c.wait()
            if ch + 2 < n_chunks:
                gather_copies[ch + 2] = start_gathers(ch + 2)

        for c in wb_copies[n_chunks - 1]:
            c.wait()

    return emb_kernel


def kernel(x, emb_table, pos_enc):
    B, S = x.shape
    V, D = emb_table.shape
    x = x.astype(jnp.int32)
    emb = _make_sc_kernel(B, S, V, D)
    return emb(x, emb_table, pos_enc)
